# baseline (speedup 1.0000x reference)
import functools
import math

import jax
import jax.numpy as jnp
from jax.experimental import pallas as pl
from jax.experimental.pallas import tpu as pltpu

N0 = 10000
RATIO = 0.9


def _round_up(x, m):
    return (x + m - 1) // m - 0 if x % m == 0 else (x // m + 1)


def _pad_rows(a, m_pad):
    m = a.shape[0]
    if m == m_pad:
        return a
    return jnp.pad(a, ((0, m_pad - m),) + ((0, 0),) * (a.ndim - 1))


def _mm_kernel(a_ref, w_ref, o_ref, *, relu):
    out = jnp.dot(a_ref[...], w_ref[...], preferred_element_type=jnp.float32)
    if relu:
        out = jnp.maximum(out, 0.0)
    o_ref[...] = out


def _matmul(a, w, *, relu=False, bm=512):
    m, k = a.shape
    k2, n = w.shape
    assert k == k2
    m_pad = ((m + bm - 1) // bm) * bm
    a_p = _pad_rows(a, m_pad)
    grid = (m_pad // bm,)
    out = pl.pallas_call(
        functools.partial(_mm_kernel, relu=relu),
        grid=grid,
        in_specs=[
            pl.BlockSpec((bm, k), lambda i: (i, 0)),
            pl.BlockSpec((k, n), lambda i: (0, 0)),
        ],
        out_specs=pl.BlockSpec((bm, n), lambda i: (i, 0)),
        out_shape=jax.ShapeDtypeStruct((m_pad, n), jnp.float32),
    )(a_p, w)
    return out[:m]


def _gcn(x, src, dst, ew, W, b):
    n = x.shape[0]
    h = _matmul(x, W)
    deg = jnp.zeros((n,), x.dtype).at[dst].add(ew) + 1.0
    dis = jax.lax.rsqrt(deg)
    hs = dis[:, None] * h
    agg = jnp.zeros_like(h).at[dst].add((ew * dis[src] * dis[dst])[:, None] * h[src])
    out = dis[:, None] * hs + agg + b
    return jnp.maximum(out, 0.0)


def _topk_pool(x, src, dst, ew, p, ratio):
    n = x.shape[0]
    k = int(math.ceil(ratio * n))
    score = jnp.tanh((x @ p) / jnp.linalg.norm(p))
    vals, perm = jax.lax.top_k(score, k)
    x_new = x[perm] * vals[:, None]
    mapping = jnp.full((n,), -1, dtype=src.dtype).at[perm].set(
        jnp.arange(k, dtype=src.dtype))
    ns = mapping[src]
    nd = mapping[dst]
    valid = (ns >= 0) & (nd >= 0) & (ew > 0)
    ns = jnp.where(valid, ns, 0)
    nd = jnp.where(valid, nd, 0)
    return x_new, ns, nd, valid.astype(x.dtype)


def kernel(x, edge_index, W1, b1, W2, b2, W3, b3, p1, W4, b4, W5, b5, W6, b6,
           p2, Wlin, blin):
    src = edge_index[0]
    dst = edge_index[1]
    ew = jnp.ones(src.shape, jnp.float32)
    h = _gcn(x, src, dst, ew, W1, b1)
    h = _gcn(h, src, dst, ew, W2, b2)
    h = _gcn(h, src, dst, ew, W3, b3)
    h, src, dst, ew = _topk_pool(h, src, dst, ew, p1, RATIO)
    h = _gcn(h, src, dst, ew, W4, b4)
    h = _gcn(h, src, dst, ew, W5, b5)
    h = _gcn(h, src, dst, ew, W6, b6)
    h, src, dst, ew = _topk_pool(h, src, dst, ew, p2, RATIO)
    g = jnp.mean(h, axis=0, keepdims=True)
    out = _matmul(g, Wlin) + blin
    return jax.nn.log_softmax(out, axis=1)


# speedup vs baseline: 3.3489x; 3.3489x over previous
import functools
import math

import jax
import jax.numpy as jnp
from jax import lax
from jax.experimental import pallas as pl
from jax.experimental.pallas import tpu as pltpu
from jax.experimental.pallas import tpu_sc as plsc

RATIO = 0.9

_NC = 2
_NS = 16
_WIN = 128
_JUNK = 128
_ZB = 64


def _pad_rows(a, m_pad):
    m = a.shape[0]
    if m == m_pad:
        return a
    return jnp.pad(a, ((0, m_pad - m),) + ((0, 0),) * (a.ndim - 1))


def _mm_kernel(a_ref, w_ref, o_ref, *, relu):
    out = jnp.dot(a_ref[...], w_ref[...], preferred_element_type=jnp.float32)
    if relu:
        out = jnp.maximum(out, 0.0)
    o_ref[...] = out


def _matmul(a, w, *, relu=False, bm=512):
    m, k = a.shape
    _, n = w.shape
    m_pad = ((m + bm - 1) // bm) * bm
    a_p = _pad_rows(a, m_pad)
    out = pl.pallas_call(
        functools.partial(_mm_kernel, relu=relu),
        grid=(m_pad // bm,),
        in_specs=[
            pl.BlockSpec((bm, k), lambda i: (i, 0)),
            pl.BlockSpec((k, n), lambda i: (0, 0)),
        ],
        out_specs=pl.BlockSpec((bm, n), lambda i: (i, 0)),
        out_shape=jax.ShapeDtypeStruct((m_pad, n), jnp.float32),
    )(a_p, w)
    return out[:m]


def _table_kernel(dis_ref, a_ref, w_ref, o_ref, *, scale_a):
    a = a_ref[...]
    if scale_a:
        a = a * dis_ref[...][:, None]
        out = jnp.dot(a, w_ref[...], preferred_element_type=jnp.float32)
    else:
        out = jnp.dot(a, w_ref[...], preferred_element_type=jnp.float32)
        out = out * dis_ref[...][:, None]
    o_ref[0] = out


def _mm_table(a, w, dis, cw, *, ls=None, bm=512):
    m_pad, k = a.shape
    n = w.shape[1]
    ncc = n // cw
    if ls is not None:
        a = a * ls[:, None]
    out = pl.pallas_call(
        functools.partial(_table_kernel, scale_a=False),
        grid=(ncc, m_pad // bm),
        in_specs=[
            pl.BlockSpec((bm,), lambda c, i: (i,)),
            pl.BlockSpec((bm, k), lambda c, i: (i, 0)),
            pl.BlockSpec((k, cw), lambda c, i: (0, c)),
        ],
        out_specs=pl.BlockSpec((1, bm, cw), lambda c, i: (c, i, 0)),
        out_shape=jax.ShapeDtypeStruct((ncc, m_pad, cw), jnp.float32),
    )(dis, a, w)
    return out


def _combine_kernel(dis_ref, p_ref, hs_ref, b_ref, o_ref, *,
                    bm, n_valid):
    dis = dis_ref[...][:, None]
    o = dis * (p_ref[0, 0, 0] + p_ref[1, 0, 0] + hs_ref[0]) + b_ref[0]
    o = jnp.maximum(o, 0.0)
    row = pl.program_id(1) * bm + lax.broadcasted_iota(jnp.int32, (bm, 1), 0)
    o_ref[...] = jnp.where(row < n_valid, o, 0.0)


def _combine(partials, hs, dis, b, n_valid, *, bm=512):
    ncc, m_pad, cw = hs.shape
    b2 = b.reshape(ncc, 1, cw)
    out = pl.pallas_call(
        functools.partial(_combine_kernel, bm=bm, n_valid=n_valid),
        grid=(ncc, m_pad // bm),
        in_specs=[
            pl.BlockSpec((bm,), lambda c, i: (i,)),
            pl.BlockSpec((2, 1, bm, cw), lambda c, i: (0, c, i, 0)),
            pl.BlockSpec((1, bm, cw), lambda c, i: (c, i, 0)),
            pl.BlockSpec((1, 1, cw), lambda c, i: (c, 0, 0)),
        ],
        out_specs=pl.BlockSpec((bm, cw), lambda c, i: (i, c)),
        out_shape=jax.ShapeDtypeStruct((m_pad, ncc * cw), jnp.float32),
    )(dis, partials, hs, b2)
    return out


def _sc_aggregate(table, src, dst, n_pad, ncc, cw):
    ep = src.shape[0]
    ept = ep // (_NC * _NS)
    nw = ept // _WIN
    rpt = n_pad // _NS
    nz = rpt // _ZB
    mesh = plsc.VectorSubcoreMesh(core_axis_name="c", subcore_axis_name="s")

    @functools.partial(
        pl.kernel,
        out_type=jax.ShapeDtypeStruct((_NC, ncc, n_pad, cw), jnp.float32),
        mesh=mesh,
        scratch_types=[
            pltpu.VMEM((_WIN,), jnp.int32),
            pltpu.VMEM((_WIN,), jnp.int32),
            pltpu.VMEM((_WIN,), jnp.int32),
            pltpu.VMEM((_WIN, cw), jnp.float32),
            pltpu.VMEM((_ZB, cw), jnp.float32),
            pltpu.VMEM((_ZB, cw), jnp.float32),
            pltpu.VMEM_SHARED((n_pad, cw), jnp.float32),
            pltpu.SemaphoreType.DMA,
        ],
    )
    def agg(table_h, src_h, dst_h, out_h, src_v, dst_v, srco_v, rows_v,
            zeros_v, fl_v, acc_s, sem):
        c = lax.axis_index("c")
        s = lax.axis_index("s")
        tid = c * _NS + s
        zsplat = jnp.zeros((16,), jnp.float32)

        def zrow(r, _):
            for v in range(cw // 16):
                zeros_v[r, pl.ds(v * 16, 16)] = zsplat
            return 0

        lax.fori_loop(0, _ZB, zrow, 0)

        def chunk_body(ch, _):
            for j in range(nz):
                pltpu.sync_copy(zeros_v,
                                acc_s.at[pl.ds(s * rpt + j * _ZB, _ZB)])
            plsc.subcore_barrier()
            coff = ch * n_pad

            def win_body(w, _):
                off = tid * ept + w * _WIN
                pltpu.sync_copy(src_h.at[pl.ds(off, _WIN)], src_v)
                pltpu.sync_copy(dst_h.at[pl.ds(off, _WIN)], dst_v)
                for v in range(_WIN // 16):
                    srco_v[pl.ds(v * 16, 16)] = (
                        src_v[pl.ds(v * 16, 16)] + coff)
                pltpu.async_copy(table_h.at[srco_v], rows_v, sem).wait()
                pltpu.sync_copy(rows_v, acc_s.at[dst_v], add=True)
                return 0

            lax.fori_loop(0, nw, win_body, 0)
            plsc.subcore_barrier()
            for j in range(nz):
                pltpu.sync_copy(acc_s.at[pl.ds(s * rpt + j * _ZB, _ZB)], fl_v)
                pltpu.sync_copy(
                    fl_v, out_h.at[c, ch, pl.ds(s * rpt + j * _ZB, _ZB)])
            plsc.subcore_barrier()
            return 0

        lax.fori_loop(0, ncc, chunk_body, 0)

    return agg(table, src, dst)


def _sc_degree(dst, n_pad):
    ep = dst.shape[0]
    ept = ep // (_NC * _NS)
    nw = ept // _WIN
    rpt = n_pad // _NS
    nz = rpt // _ZB
    mesh = plsc.VectorSubcoreMesh(core_axis_name="c", subcore_axis_name="s")

    @functools.partial(
        pl.kernel,
        out_type=jax.ShapeDtypeStruct((_NC * n_pad,), jnp.float32),
        mesh=mesh,
        scratch_types=[
            pltpu.VMEM((_WIN,), jnp.int32),
            pltpu.VMEM((_WIN,), jnp.float32),
            pltpu.VMEM((_ZB,), jnp.float32),
            pltpu.VMEM((_ZB,), jnp.float32),
            pltpu.VMEM_SHARED((n_pad,), jnp.float32),
            pltpu.SemaphoreType.DMA,
        ],
    )
    def deg(dst_h, out_h, dst_v, ones_v, zeros_v, fl_v, acc_s, sem):
        c = lax.axis_index("c")
        s = lax.axis_index("s")
        tid = c * _NS + s
        for v in range(_WIN // 16):
            ones_v[pl.ds(v * 16, 16)] = jnp.ones((16,), jnp.float32)
        for v in range(_ZB // 16):
            zeros_v[pl.ds(v * 16, 16)] = jnp.zeros((16,), jnp.float32)
        for j in range(nz):
            pltpu.sync_copy(zeros_v, acc_s.at[pl.ds(s * rpt + j * _ZB, _ZB)])
        plsc.subcore_barrier()

        def win_body(w, _):
            off = tid * ept + w * _WIN
            pltpu.sync_copy(dst_h.at[pl.ds(off, _WIN)], dst_v)
            pltpu.sync_copy(ones_v, acc_s.at[dst_v], add=True)
            return 0

        lax.fori_loop(0, nw, win_body, 0)
        plsc.subcore_barrier()
        for j in range(nz):
            pltpu.sync_copy(acc_s.at[pl.ds(s * rpt + j * _ZB, _ZB)], fl_v)
            pltpu.sync_copy(
                fl_v, out_h.at[pl.ds(c * n_pad + s * rpt + j * _ZB, _ZB)])

    return deg(dst).reshape(_NC, n_pad)


def _gcn_layer(h, src, dst, dis, W, b, n, n_pad, cw, *, ls=None):
    cout = W.shape[1]
    if cout % cw:
        cpad = ((cout + cw - 1) // cw) * cw
        W = jnp.pad(W, ((0, 0), (0, cpad - cout)))
        b = jnp.pad(b, (0, cpad - cout))
    hs = _mm_table(h, W, dis, cw, ls=ls)
    ncc = hs.shape[0]
    table = hs.reshape(ncc * n_pad, cw)
    partials = _sc_aggregate(table, src, dst, n_pad, ncc, cw)
    return _combine(partials, hs, dis, b, n)


def _pad_edges(src, dst, junk_base, ep):
    e = src.shape[0]
    pad = ep - e
    fill = junk_base + (jnp.arange(pad, dtype=jnp.int32) % _JUNK)
    return (jnp.concatenate([src, fill]), jnp.concatenate([dst, fill]))


def kernel(x, edge_index, W1, b1, W2, b2, W3, b3, p1, W4, b4, W5, b5, W6, b6,
           p2, Wlin, blin):
    n1 = x.shape[0]
    e = edge_index.shape[1]
    k1 = int(math.ceil(RATIO * n1))
    k2 = int(math.ceil(RATIO * k1))
    npad1 = ((n1 + _JUNK) + 1023) // 1024 * 1024
    npad2 = ((k1 + _JUNK) + 1023) // 1024 * 1024
    ep = ((e + (_NC * _NS * _WIN) - 1) // (_NC * _NS * _WIN)) * (_NC * _NS * _WIN)

    src0 = edge_index[0]
    dst0 = edge_index[1]
    src1, dst1 = _pad_edges(src0, dst0, n1, ep)

    degp1 = _sc_degree(dst1, npad1)
    deg1 = degp1[0] + degp1[1] + 1.0
    dis1 = lax.rsqrt(deg1)

    xp = _pad_rows(x, npad1)
    h = _gcn_layer(xp, src1, dst1, dis1, W1, b1, n1, npad1, 128)
    h = _gcn_layer(h, src1, dst1, dis1, W2, b2, n1, npad1, 128)
    h = _gcn_layer(h, src1, dst1, dis1, W3, b3, n1, npad1, 128)

    h_valid = h[:n1]
    score = jnp.tanh((h_valid @ p1) / jnp.linalg.norm(p1))
    vals, perm = lax.top_k(score, k1)
    x_new = h_valid[perm]
    mapping = jnp.full((n1,), -1, jnp.int32).at[perm].set(
        jnp.arange(k1, dtype=jnp.int32))
    ns = mapping[src0]
    nd = mapping[dst0]
    valid = (ns >= 0) & (nd >= 0)
    junk = k1 + (jnp.arange(e, dtype=jnp.int32) % _JUNK)
    ns = jnp.where(valid, ns, junk)
    nd = jnp.where(valid, nd, junk)
    src2, dst2 = _pad_edges(ns, nd, k1, ep)

    degp2 = _sc_degree(dst2, npad2)
    deg2 = degp2[0] + degp2[1] + 1.0
    dis2 = lax.rsqrt(deg2)

    xg = _pad_rows(x_new, npad2)
    lsv = _pad_rows(vals[:, None], npad2)[:, 0]
    h = _gcn_layer(xg, src2, dst2, dis2, W4, b4, k1, npad2, 128, ls=lsv)
    h = _gcn_layer(h, src2, dst2, dis2, W5, b5, k1, npad2, 128)
    h = _gcn_layer(h, src2, dst2, dis2, W6, b6, k1, npad2, 128)

    h_valid = h[:k1]
    score2 = jnp.tanh((h_valid @ p2) / jnp.linalg.norm(p2))
    vals2, perm2 = lax.top_k(score2, k2)
    w_full = jnp.zeros((k1,), jnp.float32).at[perm2].set(vals2)
    g = (w_full @ h_valid)[None, :] / k2
    out = _matmul(g, Wlin) + blin
    return jax.nn.log_softmax(out, axis=1)


# speedup vs baseline: 4.1212x; 1.2306x over previous
import functools
import math

import jax
import jax.numpy as jnp
from jax import lax
from jax.experimental import pallas as pl
from jax.experimental.pallas import tpu as pltpu
from jax.experimental.pallas import tpu_sc as plsc

RATIO = 0.9

_NC = 2
_NS = 16
_WIN = 128
_JUNK = 128
_ZB = 64


def _pad_rows(a, m_pad):
    m = a.shape[0]
    if m == m_pad:
        return a
    return jnp.pad(a, ((0, m_pad - m),) + ((0, 0),) * (a.ndim - 1))


def _mm_kernel(a_ref, w_ref, o_ref, *, relu):
    out = jnp.dot(a_ref[...], w_ref[...], preferred_element_type=jnp.float32)
    if relu:
        out = jnp.maximum(out, 0.0)
    o_ref[...] = out


def _matmul(a, w, *, relu=False, bm=512):
    m, k = a.shape
    _, n = w.shape
    m_pad = ((m + bm - 1) // bm) * bm
    a_p = _pad_rows(a, m_pad)
    out = pl.pallas_call(
        functools.partial(_mm_kernel, relu=relu),
        grid=(m_pad // bm,),
        in_specs=[
            pl.BlockSpec((bm, k), lambda i: (i, 0)),
            pl.BlockSpec((k, n), lambda i: (0, 0)),
        ],
        out_specs=pl.BlockSpec((bm, n), lambda i: (i, 0)),
        out_shape=jax.ShapeDtypeStruct((m_pad, n), jnp.float32),
    )(a_p, w)
    return out[:m]


def _table_kernel(dis_ref, a_ref, w_ref, o_ref, *, scale_a):
    a = a_ref[...]
    if scale_a:
        a = a * dis_ref[...][:, None]
        out = jnp.dot(a, w_ref[...], preferred_element_type=jnp.float32)
    else:
        out = jnp.dot(a, w_ref[...], preferred_element_type=jnp.float32)
        out = out * dis_ref[...][:, None]
    o_ref[0] = out


def _mm_table(a, w, dis, cw, *, ls=None, bm=512):
    m_pad, k = a.shape
    n = w.shape[1]
    ncc = n // cw
    if ls is not None:
        a = a * ls[:, None]
    out = pl.pallas_call(
        functools.partial(_table_kernel, scale_a=False),
        grid=(ncc, m_pad // bm),
        in_specs=[
            pl.BlockSpec((bm,), lambda c, i: (i,)),
            pl.BlockSpec((bm, k), lambda c, i: (i, 0)),
            pl.BlockSpec((k, cw), lambda c, i: (0, c)),
        ],
        out_specs=pl.BlockSpec((1, bm, cw), lambda c, i: (c, i, 0)),
        out_shape=jax.ShapeDtypeStruct((ncc, m_pad, cw), jnp.float32),
    )(dis, a, w)
    return out


def _combine_kernel(dis_ref, p_ref, hs_ref, b_ref, o_ref, *,
                    bm, n_valid):
    dis = dis_ref[...][:, None]
    o = dis * (p_ref[0, 0, 0] + p_ref[1, 0, 0] + hs_ref[0]) + b_ref[0]
    o = jnp.maximum(o, 0.0)
    row = pl.program_id(1) * bm + lax.broadcasted_iota(jnp.int32, (bm, 1), 0)
    o_ref[...] = jnp.where(row < n_valid, o, 0.0)


def _combine(partials, hs, dis, b, n_valid, *, bm=512):
    ncc, m_pad, cw = hs.shape
    b2 = b.reshape(ncc, 1, cw)
    out = pl.pallas_call(
        functools.partial(_combine_kernel, bm=bm, n_valid=n_valid),
        grid=(ncc, m_pad // bm),
        in_specs=[
            pl.BlockSpec((bm,), lambda c, i: (i,)),
            pl.BlockSpec((2, 1, bm, cw), lambda c, i: (0, c, i, 0)),
            pl.BlockSpec((1, bm, cw), lambda c, i: (c, i, 0)),
            pl.BlockSpec((1, 1, cw), lambda c, i: (c, 0, 0)),
        ],
        out_specs=pl.BlockSpec((bm, cw), lambda c, i: (i, c)),
        out_shape=jax.ShapeDtypeStruct((m_pad, ncc * cw), jnp.float32),
    )(dis, partials, hs, b2)
    return out


def _sc_aggregate(table, src, dst, n_pad, ncc, cw):
    ep = src.shape[0]
    ept = ep // (_NC * _NS)
    nw = ept // _WIN
    rpt = n_pad // _NS
    nz = rpt // _ZB
    mesh = plsc.VectorSubcoreMesh(core_axis_name="c", subcore_axis_name="s")

    assert nw % 2 == 0
    @functools.partial(
        pl.kernel,
        out_type=jax.ShapeDtypeStruct((_NC, ncc, n_pad, cw), jnp.float32),
        mesh=mesh,
        scratch_types=[
            pltpu.VMEM((_WIN,), jnp.int32),
            pltpu.VMEM((_WIN,), jnp.int32),
            pltpu.VMEM((_WIN,), jnp.int32),
            pltpu.VMEM((_WIN,), jnp.int32),
            pltpu.VMEM((_WIN,), jnp.int32),
            pltpu.VMEM((_WIN,), jnp.int32),
            pltpu.VMEM((_WIN, cw), jnp.float32),
            pltpu.VMEM((_WIN, cw), jnp.float32),
            pltpu.VMEM_SHARED((n_pad, cw), jnp.float32),
            pltpu.SemaphoreType.DMA,
            pltpu.SemaphoreType.DMA,
        ],
    )
    def agg(table_h, src_h, dst_h, out_h, src_v0, dst_v0, srco_v0,
            src_v1, dst_v1, srco_v1, rows_v0, rows_v1,
            acc_s, sem0, sem1):
        c = lax.axis_index("c")
        s = lax.axis_index("s")
        tid = c * _NS + s
        zsplat = jnp.zeros((16,), jnp.float32)
        bufs = ((src_v0, dst_v0, srco_v0, rows_v0, sem0),
                (src_v1, dst_v1, srco_v1, rows_v1, sem1))

        def zrow(r, _):
            for v in range(cw // 16):
                rows_v1[r, pl.ds(v * 16, 16)] = zsplat
            return 0

        def chunk_body(ch, _):
            lax.fori_loop(0, _ZB, zrow, 0)
            for j in range(nz):
                pltpu.sync_copy(rows_v1.at[pl.ds(0, _ZB)],
                                acc_s.at[pl.ds(s * rpt + j * _ZB, _ZB)])
            plsc.subcore_barrier()
            coff = ch * n_pad

            def start_gather(w, p):
                sv, dv, ov, rv, sm = bufs[p]
                off = tid * ept + w * _WIN
                pltpu.sync_copy(src_h.at[pl.ds(off, _WIN)], sv)
                pltpu.sync_copy(dst_h.at[pl.ds(off, _WIN)], dv)
                for v in range(_WIN // 16):
                    ov[pl.ds(v * 16, 16)] = sv[pl.ds(v * 16, 16)] + coff
                pltpu.async_copy(table_h.at[ov], rv, sm)

            def finish(p):
                sv, dv, ov, rv, sm = bufs[p]
                pltpu.make_async_copy(table_h.at[ov], rv, sm).wait()
                pltpu.sync_copy(rv, acc_s.at[dv], add=True)

            start_gather(0, 0)

            def win_body(t, _):
                w0 = 2 * t
                start_gather(w0 + 1, 1)
                finish(0)

                @pl.when(w0 + 2 < nw)
                def _():
                    start_gather(w0 + 2, 0)

                finish(1)
                return 0

            lax.fori_loop(0, nw // 2, win_body, 0)
            plsc.subcore_barrier()
            fl_v = rows_v0.at[pl.ds(0, _ZB)]
            for j in range(nz):
                pltpu.sync_copy(acc_s.at[pl.ds(s * rpt + j * _ZB, _ZB)], fl_v)
                pltpu.sync_copy(
                    fl_v, out_h.at[c, ch, pl.ds(s * rpt + j * _ZB, _ZB)])
            plsc.subcore_barrier()
            return 0

        lax.fori_loop(0, ncc, chunk_body, 0)

    return agg(table, src, dst)


def _sc_degree(dst, n_pad):
    ep = dst.shape[0]
    ept = ep // (_NC * _NS)
    nw = ept // _WIN
    rpt = n_pad // _NS
    nz = rpt // _ZB
    mesh = plsc.VectorSubcoreMesh(core_axis_name="c", subcore_axis_name="s")

    @functools.partial(
        pl.kernel,
        out_type=jax.ShapeDtypeStruct((_NC * n_pad,), jnp.float32),
        mesh=mesh,
        scratch_types=[
            pltpu.VMEM((_WIN,), jnp.int32),
            pltpu.VMEM((_WIN,), jnp.float32),
            pltpu.VMEM((_ZB,), jnp.float32),
            pltpu.VMEM((_ZB,), jnp.float32),
            pltpu.VMEM_SHARED((n_pad,), jnp.float32),
            pltpu.SemaphoreType.DMA,
        ],
    )
    def deg(dst_h, out_h, dst_v, ones_v, zeros_v, fl_v, acc_s, sem):
        c = lax.axis_index("c")
        s = lax.axis_index("s")
        tid = c * _NS + s
        for v in range(_WIN // 16):
            ones_v[pl.ds(v * 16, 16)] = jnp.ones((16,), jnp.float32)
        for v in range(_ZB // 16):
            zeros_v[pl.ds(v * 16, 16)] = jnp.zeros((16,), jnp.float32)
        for j in range(nz):
            pltpu.sync_copy(zeros_v, acc_s.at[pl.ds(s * rpt + j * _ZB, _ZB)])
        plsc.subcore_barrier()

        def win_body(w, _):
            off = tid * ept + w * _WIN
            pltpu.sync_copy(dst_h.at[pl.ds(off, _WIN)], dst_v)
            pltpu.sync_copy(ones_v, acc_s.at[dst_v], add=True)
            return 0

        lax.fori_loop(0, nw, win_body, 0)
        plsc.subcore_barrier()
        for j in range(nz):
            pltpu.sync_copy(acc_s.at[pl.ds(s * rpt + j * _ZB, _ZB)], fl_v)
            pltpu.sync_copy(
                fl_v, out_h.at[pl.ds(c * n_pad + s * rpt + j * _ZB, _ZB)])

    return deg(dst).reshape(_NC, n_pad)


def _gcn_layer(h, src, dst, dis, W, b, n, n_pad, cw, *, ls=None):
    cout = W.shape[1]
    if cout % cw:
        cpad = ((cout + cw - 1) // cw) * cw
        W = jnp.pad(W, ((0, 0), (0, cpad - cout)))
        b = jnp.pad(b, (0, cpad - cout))
    hs = _mm_table(h, W, dis, cw, ls=ls)
    ncc = hs.shape[0]
    table = hs.reshape(ncc * n_pad, cw)
    partials = _sc_aggregate(table, src, dst, n_pad, ncc, cw)
    return _combine(partials, hs, dis, b, n)


def _pad_edges(src, dst, junk_base, ep):
    e = src.shape[0]
    pad = ep - e
    fill = junk_base + (jnp.arange(pad, dtype=jnp.int32) % _JUNK)
    return (jnp.concatenate([src, fill]), jnp.concatenate([dst, fill]))


def kernel(x, edge_index, W1, b1, W2, b2, W3, b3, p1, W4, b4, W5, b5, W6, b6,
           p2, Wlin, blin):
    n1 = x.shape[0]
    e = edge_index.shape[1]
    k1 = int(math.ceil(RATIO * n1))
    k2 = int(math.ceil(RATIO * k1))
    npad1 = ((n1 + _JUNK) + 1023) // 1024 * 1024
    npad2 = ((k1 + _JUNK) + 1023) // 1024 * 1024
    epm = _NC * _NS * _WIN * 2
    ep = ((e + epm - 1) // epm) * epm

    src0 = edge_index[0]
    dst0 = edge_index[1]
    src1, dst1 = _pad_edges(src0, dst0, n1, ep)

    degp1 = _sc_degree(dst1, npad1)
    deg1 = degp1[0] + degp1[1] + 1.0
    dis1 = lax.rsqrt(deg1)

    xp = _pad_rows(x, npad1)
    h = _gcn_layer(xp, src1, dst1, dis1, W1, b1, n1, npad1, 128)
    h = _gcn_layer(h, src1, dst1, dis1, W2, b2, n1, npad1, 128)
    h = _gcn_layer(h, src1, dst1, dis1, W3, b3, n1, npad1, 128)

    h_valid = h[:n1]
    score = jnp.tanh((h_valid @ p1) / jnp.linalg.norm(p1))
    vals, perm = lax.top_k(score, k1)
    x_new = h_valid[perm]
    mapping = jnp.full((n1,), -1, jnp.int32).at[perm].set(
        jnp.arange(k1, dtype=jnp.int32))
    ns = mapping[src0]
    nd = mapping[dst0]
    valid = (ns >= 0) & (nd >= 0)
    junk = k1 + (jnp.arange(e, dtype=jnp.int32) % _JUNK)
    ns = jnp.where(valid, ns, junk)
    nd = jnp.where(valid, nd, junk)
    src2, dst2 = _pad_edges(ns, nd, k1, ep)

    degp2 = _sc_degree(dst2, npad2)
    deg2 = degp2[0] + degp2[1] + 1.0
    dis2 = lax.rsqrt(deg2)

    xg = _pad_rows(x_new, npad2)
    lsv = _pad_rows(vals[:, None], npad2)[:, 0]
    h = _gcn_layer(xg, src2, dst2, dis2, W4, b4, k1, npad2, 128, ls=lsv)
    h = _gcn_layer(h, src2, dst2, dis2, W5, b5, k1, npad2, 128)
    h = _gcn_layer(h, src2, dst2, dis2, W6, b6, k1, npad2, 128)

    h_valid = h[:k1]
    score2 = jnp.tanh((h_valid @ p2) / jnp.linalg.norm(p2))
    vals2, perm2 = lax.top_k(score2, k2)
    w_full = jnp.zeros((k1,), jnp.float32).at[perm2].set(vals2)
    g = (w_full @ h_valid)[None, :] / k2
    out = _matmul(g, Wlin) + blin
    return jax.nn.log_softmax(out, axis=1)


# speedup vs baseline: 4.3040x; 1.0443x over previous
import functools
import math

import jax
import jax.numpy as jnp
from jax import lax
from jax.experimental import pallas as pl
from jax.experimental.pallas import tpu as pltpu
from jax.experimental.pallas import tpu_sc as plsc

RATIO = 0.9

_NC = 2
_NS = 16
_WIN = 128
_JUNK = 128
_ZB = 64


def _pad_rows(a, m_pad):
    m = a.shape[0]
    if m == m_pad:
        return a
    return jnp.pad(a, ((0, m_pad - m),) + ((0, 0),) * (a.ndim - 1))


def _mm_kernel(a_ref, w_ref, o_ref, *, relu):
    out = jnp.dot(a_ref[...], w_ref[...], preferred_element_type=jnp.float32)
    if relu:
        out = jnp.maximum(out, 0.0)
    o_ref[...] = out


def _matmul(a, w, *, relu=False, bm=512):
    m, k = a.shape
    _, n = w.shape
    m_pad = ((m + bm - 1) // bm) * bm
    a_p = _pad_rows(a, m_pad)
    out = pl.pallas_call(
        functools.partial(_mm_kernel, relu=relu),
        grid=(m_pad // bm,),
        in_specs=[
            pl.BlockSpec((bm, k), lambda i: (i, 0)),
            pl.BlockSpec((k, n), lambda i: (0, 0)),
        ],
        out_specs=pl.BlockSpec((bm, n), lambda i: (i, 0)),
        out_shape=jax.ShapeDtypeStruct((m_pad, n), jnp.float32),
    )(a_p, w)
    return out[:m]


def _table_kernel(dis_ref, a_ref, w_ref, o_ref, *, dtype):
    out = jnp.dot(a_ref[...], w_ref[...], preferred_element_type=jnp.float32)
    out = out * dis_ref[...][:, None]
    o_ref[0] = out.astype(dtype)


def _mm_table(a, w, dis, cw, dtype, *, ls=None, bm=512):
    m_pad, k = a.shape
    n = w.shape[1]
    ncc = n // cw
    if ls is not None:
        a = a * ls[:, None]
    out = pl.pallas_call(
        functools.partial(_table_kernel, dtype=dtype),
        grid=(ncc, m_pad // bm),
        in_specs=[
            pl.BlockSpec((bm,), lambda c, i: (i,)),
            pl.BlockSpec((bm, k), lambda c, i: (i, 0)),
            pl.BlockSpec((k, cw), lambda c, i: (0, c)),
        ],
        out_specs=pl.BlockSpec((1, bm, cw), lambda c, i: (c, i, 0)),
        out_shape=jax.ShapeDtypeStruct((ncc, m_pad, cw), dtype),
    )(dis, a, w)
    return out


def _combine_kernel(dis_ref, p_ref, hs_ref, b_ref, o_ref, *,
                    bm, n_valid):
    dis = dis_ref[...][:, None]
    psum = (p_ref[0, 0, 0].astype(jnp.float32)
            + p_ref[1, 0, 0].astype(jnp.float32)
            + hs_ref[0].astype(jnp.float32))
    o = dis * psum + b_ref[0]
    o = jnp.maximum(o, 0.0)
    row = pl.program_id(1) * bm + lax.broadcasted_iota(jnp.int32, (bm, 1), 0)
    o_ref[...] = jnp.where(row < n_valid, o, 0.0)


def _combine(partials, hs, dis, b, n_valid, *, bm=512):
    ncc, m_pad, cw = hs.shape
    b2 = b.reshape(ncc, 1, cw)
    out = pl.pallas_call(
        functools.partial(_combine_kernel, bm=bm, n_valid=n_valid),
        grid=(ncc, m_pad // bm),
        in_specs=[
            pl.BlockSpec((bm,), lambda c, i: (i,)),
            pl.BlockSpec((2, 1, bm, cw), lambda c, i: (0, c, i, 0)),
            pl.BlockSpec((1, bm, cw), lambda c, i: (c, i, 0)),
            pl.BlockSpec((1, 1, cw), lambda c, i: (c, 0, 0)),
        ],
        out_specs=pl.BlockSpec((bm, cw), lambda c, i: (i, c)),
        out_shape=jax.ShapeDtypeStruct((m_pad, ncc * cw), jnp.float32),
    )(dis, partials, hs, b2)
    return out


def _sc_aggregate(table, src, dst, n_pad, ncc, cw):
    ep = src.shape[0]
    ept = ep // (_NC * _NS)
    nw = ept // _WIN
    rpt = n_pad // _NS
    nz = rpt // _ZB
    mesh = plsc.VectorSubcoreMesh(core_axis_name="c", subcore_axis_name="s")

    assert nw % 2 == 0 and nw >= 4
    @functools.partial(
        pl.kernel,
        out_type=jax.ShapeDtypeStruct((_NC, ncc, n_pad, cw), jnp.float32),
        mesh=mesh,
        scratch_types=[
            pltpu.VMEM((_WIN,), jnp.int32),
            pltpu.VMEM((_WIN,), jnp.int32),
            pltpu.VMEM((_WIN,), jnp.int32),
            pltpu.VMEM((_WIN,), jnp.int32),
            pltpu.VMEM((_WIN,), jnp.int32),
            pltpu.VMEM((_WIN,), jnp.int32),
            pltpu.VMEM((_WIN,), jnp.int32),
            pltpu.VMEM((_WIN,), jnp.int32),
            pltpu.VMEM((_WIN, cw), jnp.float32),
            pltpu.VMEM((_WIN, cw), jnp.float32),
            pltpu.VMEM_SHARED((n_pad, cw), jnp.float32),
            pltpu.SemaphoreType.DMA,
            pltpu.SemaphoreType.DMA,
            pltpu.SemaphoreType.DMA,
            pltpu.SemaphoreType.DMA,
            pltpu.SemaphoreType.DMA,
            pltpu.SemaphoreType.DMA,
        ],
    )
    def agg(table_h, src_h, dst_h, out_h, src_v0, dst_v0, srco_v0, sdst_v0,
            src_v1, dst_v1, srco_v1, sdst_v1, rows_v0, rows_v1,
            acc_s, isem0, isem1, gsem0, gsem1, ssem0, ssem1):
        c = lax.axis_index("c")
        s = lax.axis_index("s")
        tid = c * _NS + s
        zsplat = jnp.zeros((16,), jnp.float32)
        idx = ((src_v0, dst_v0, srco_v0, sdst_v0, isem0),
               (src_v1, dst_v1, srco_v1, sdst_v1, isem1))
        rows = (rows_v0, rows_v1)
        gsems = (gsem0, gsem1)
        ssems = (ssem0, ssem1)

        def zrow(r, _):
            for v in range(cw // 16):
                rows_v1[r, pl.ds(v * 16, 16)] = zsplat
            return 0

        def issue_idx(w, p):
            sv, dv = idx[p][0], idx[p][1]
            off = tid * ept + w * _WIN
            pltpu.async_copy(src_h.at[pl.ds(off, _WIN)], sv, idx[p][4])
            pltpu.async_copy(dst_h.at[pl.ds(off, _WIN)], dv, idx[p][4])

        def wait_idx(p):
            sv, dv = idx[p][0], idx[p][1]
            pltpu.make_async_copy(src_h.at[pl.ds(0, _WIN)], sv,
                                  idx[p][4]).wait()
            pltpu.make_async_copy(dst_h.at[pl.ds(0, _WIN)], dv,
                                  idx[p][4]).wait()

        def comp_srco(p, coff):
            sv, ov = idx[p][0], idx[p][2]
            for v in range(_WIN // 16):
                ov[pl.ds(v * 16, 16)] = sv[pl.ds(v * 16, 16)] + coff

        def copy_sdst(p):
            dv, sd = idx[p][1], idx[p][3]
            for v in range(_WIN // 16):
                sd[pl.ds(v * 16, 16)] = dv[pl.ds(v * 16, 16)]

        def issue_gather(p):
            pltpu.async_copy(table_h.at[idx[p][2]], rows[p], gsems[p])

        def wait_gather(p):
            pltpu.make_async_copy(table_h.at[idx[p][2]], rows[p],
                                  gsems[p]).wait()

        def issue_scat(p):
            pltpu.async_copy(rows[p], acc_s.at[idx[p][3]], ssems[p],
                             add=True)

        def wait_scat(p):
            pltpu.make_async_copy(rows[p], acc_s.at[idx[p][3]],
                                  ssems[p]).wait()

        def chunk_body(ch, _):
            lax.fori_loop(0, _ZB, zrow, 0)
            for j in range(nz):
                zoff = pl.multiple_of(s * rpt + j * _ZB, _ZB)
                pltpu.sync_copy(rows_v1.at[pl.ds(0, _ZB)],
                                acc_s.at[pl.ds(zoff, _ZB)])
            plsc.subcore_barrier()
            coff = ch * n_pad

            issue_idx(0, 0)
            wait_idx(0)
            comp_srco(0, coff)
            issue_gather(0)
            issue_idx(1, 1)

            def win_body(t, _):
                w0 = 2 * t

                wait_gather(0)
                copy_sdst(0)
                issue_scat(0)
                wait_idx(1)
                comp_srco(1, coff)

                @pl.when(w0 + 2 < nw)
                def _():
                    issue_idx(w0 + 2, 0)

                @pl.when(t > 0)
                def _():
                    wait_scat(1)
                issue_gather(1)

                wait_gather(1)
                copy_sdst(1)
                issue_scat(1)

                @pl.when(w0 + 2 < nw)
                def _():
                    wait_idx(0)
                    comp_srco(0, coff)

                @pl.when(w0 + 3 < nw)
                def _():
                    issue_idx(w0 + 3, 1)

                wait_scat(0)

                @pl.when(w0 + 2 < nw)
                def _():
                    issue_gather(0)

                return 0

            lax.fori_loop(0, nw // 2, win_body, 0)
            wait_scat(1)
            plsc.subcore_barrier()
            fl_v = rows_v0.at[pl.ds(0, _ZB)]
            for j in range(nz):
                foff = pl.multiple_of(s * rpt + j * _ZB, _ZB)
                pltpu.sync_copy(acc_s.at[pl.ds(foff, _ZB)], fl_v)
                pltpu.sync_copy(
                    fl_v, out_h.at[c, ch, pl.ds(foff, _ZB)])
            plsc.subcore_barrier()
            return 0

        lax.fori_loop(0, ncc, chunk_body, 0)

    return agg(table, src, dst)


def _sc_degree(dst, n_pad):
    ep = dst.shape[0]
    ept = ep // (_NC * _NS)
    nw = ept // _WIN
    rpt = n_pad // _NS
    nz = rpt // _ZB
    mesh = plsc.VectorSubcoreMesh(core_axis_name="c", subcore_axis_name="s")

    @functools.partial(
        pl.kernel,
        out_type=jax.ShapeDtypeStruct((_NC * n_pad,), jnp.float32),
        mesh=mesh,
        scratch_types=[
            pltpu.VMEM((_WIN,), jnp.int32),
            pltpu.VMEM((_WIN,), jnp.float32),
            pltpu.VMEM((_ZB,), jnp.float32),
            pltpu.VMEM((_ZB,), jnp.float32),
            pltpu.VMEM_SHARED((n_pad,), jnp.float32),
            pltpu.SemaphoreType.DMA,
        ],
    )
    def deg(dst_h, out_h, dst_v, ones_v, zeros_v, fl_v, acc_s, sem):
        c = lax.axis_index("c")
        s = lax.axis_index("s")
        tid = c * _NS + s
        for v in range(_WIN // 16):
            ones_v[pl.ds(v * 16, 16)] = jnp.ones((16,), jnp.float32)
        for v in range(_ZB // 16):
            zeros_v[pl.ds(v * 16, 16)] = jnp.zeros((16,), jnp.float32)
        for j in range(nz):
            pltpu.sync_copy(zeros_v, acc_s.at[pl.ds(s * rpt + j * _ZB, _ZB)])
        plsc.subcore_barrier()

        def win_body(w, _):
            off = tid * ept + w * _WIN
            pltpu.sync_copy(dst_h.at[pl.ds(off, _WIN)], dst_v)
            pltpu.sync_copy(ones_v, acc_s.at[dst_v], add=True)
            return 0

        lax.fori_loop(0, nw, win_body, 0)
        plsc.subcore_barrier()
        for j in range(nz):
            pltpu.sync_copy(acc_s.at[pl.ds(s * rpt + j * _ZB, _ZB)], fl_v)
            pltpu.sync_copy(
                fl_v, out_h.at[pl.ds(c * n_pad + s * rpt + j * _ZB, _ZB)])

    return deg(dst).reshape(_NC, n_pad)


def _gcn_layer(h, src, dst, dis, W, b, n, n_pad, cw, *, ls=None):
    cout = W.shape[1]
    if cout % cw:
        cpad = ((cout + cw - 1) // cw) * cw
        W = jnp.pad(W, ((0, 0), (0, cpad - cout)))
        b = jnp.pad(b, (0, cpad - cout))
    hs = _mm_table(h, W, dis, cw, jnp.float32, ls=ls)
    ncc = hs.shape[0]
    table = hs.reshape(ncc * n_pad, cw)
    partials = _sc_aggregate(table, src, dst, n_pad, ncc, cw)
    return _combine(partials, hs, dis, b, n)


def _pad_edges(src, dst, junk_base, ep):
    e = src.shape[0]
    pad = ep - e
    fill = junk_base + (jnp.arange(pad, dtype=jnp.int32) % _JUNK)
    return (jnp.concatenate([src, fill]), jnp.concatenate([dst, fill]))


def kernel(x, edge_index, W1, b1, W2, b2, W3, b3, p1, W4, b4, W5, b5, W6, b6,
           p2, Wlin, blin):
    n1 = x.shape[0]
    e = edge_index.shape[1]
    k1 = int(math.ceil(RATIO * n1))
    k2 = int(math.ceil(RATIO * k1))
    npad1 = ((n1 + _JUNK) + 1023) // 1024 * 1024
    npad2 = ((k1 + _JUNK) + 1023) // 1024 * 1024
    epm = _NC * _NS * _WIN * 2
    ep = ((e + epm - 1) // epm) * epm

    src0 = edge_index[0]
    dst0 = edge_index[1]
    src1, dst1 = _pad_edges(src0, dst0, n1, ep)

    degp1 = _sc_degree(dst1, npad1)
    deg1 = degp1[0] + degp1[1] + 1.0
    dis1 = lax.rsqrt(deg1)

    xp = _pad_rows(x, npad1)
    h = _gcn_layer(xp, src1, dst1, dis1, W1, b1, n1, npad1, 128)
    h = _gcn_layer(h, src1, dst1, dis1, W2, b2, n1, npad1, 128)
    h = _gcn_layer(h, src1, dst1, dis1, W3, b3, n1, npad1, 128)

    h_valid = h[:n1]
    score = jnp.tanh((h_valid @ p1) / jnp.linalg.norm(p1))
    vals, perm = lax.top_k(score, k1)
    x_new = h_valid[perm]
    mapping = jnp.full((n1,), -1, jnp.int32).at[perm].set(
        jnp.arange(k1, dtype=jnp.int32))
    ns = mapping[src0]
    nd = mapping[dst0]
    valid = (ns >= 0) & (nd >= 0)
    junk = k1 + (jnp.arange(e, dtype=jnp.int32) % _JUNK)
    ns = jnp.where(valid, ns, junk)
    nd = jnp.where(valid, nd, junk)
    src2, dst2 = _pad_edges(ns, nd, k1, ep)

    degp2 = _sc_degree(dst2, npad2)
    deg2 = degp2[0] + degp2[1] + 1.0
    dis2 = lax.rsqrt(deg2)

    xg = _pad_rows(x_new, npad2)
    lsv = _pad_rows(vals[:, None], npad2)[:, 0]
    h = _gcn_layer(xg, src2, dst2, dis2, W4, b4, k1, npad2, 128, ls=lsv)
    h = _gcn_layer(h, src2, dst2, dis2, W5, b5, k1, npad2, 128)
    h = _gcn_layer(h, src2, dst2, dis2, W6, b6, k1, npad2, 128)

    h_valid = h[:k1]
    score2 = jnp.tanh((h_valid @ p2) / jnp.linalg.norm(p2))
    vals2, perm2 = lax.top_k(score2, k2)
    w_full = jnp.zeros((k1,), jnp.float32).at[perm2].set(vals2)
    g = (w_full @ h_valid)[None, :] / k2
    out = _matmul(g, Wlin) + blin
    return jax.nn.log_softmax(out, axis=1)


# speedup vs baseline: 9.1876x; 2.1347x over previous
import functools
import math

import jax
import jax.numpy as jnp
from jax import lax
from jax.experimental import pallas as pl
from jax.experimental.pallas import tpu as pltpu
from jax.experimental.pallas import tpu_sc as plsc

RATIO = 0.9

_NC = 2
_NS = 16
_WIN = 128
_JUNK = 128
_ZB = 64


def _pad_rows(a, m_pad):
    m = a.shape[0]
    if m == m_pad:
        return a
    return jnp.pad(a, ((0, m_pad - m),) + ((0, 0),) * (a.ndim - 1))


def _mm_kernel(a_ref, w_ref, o_ref, *, relu):
    out = jnp.dot(a_ref[...], w_ref[...], preferred_element_type=jnp.float32)
    if relu:
        out = jnp.maximum(out, 0.0)
    o_ref[...] = out


def _matmul(a, w, *, relu=False, bm=512):
    m, k = a.shape
    _, n = w.shape
    m_pad = ((m + bm - 1) // bm) * bm
    a_p = _pad_rows(a, m_pad)
    out = pl.pallas_call(
        functools.partial(_mm_kernel, relu=relu),
        grid=(m_pad // bm,),
        in_specs=[
            pl.BlockSpec((bm, k), lambda i: (i, 0)),
            pl.BlockSpec((k, n), lambda i: (0, 0)),
        ],
        out_specs=pl.BlockSpec((bm, n), lambda i: (i, 0)),
        out_shape=jax.ShapeDtypeStruct((m_pad, n), jnp.float32),
    )(a_p, w)
    return out[:m]


def _table_kernel(dis_ref, a_ref, w_ref, o_ref, *, dtype):
    out = jnp.dot(a_ref[...], w_ref[...], preferred_element_type=jnp.float32)
    out = out * dis_ref[...][:, None]
    o_ref[0] = out.astype(dtype)


def _mm_table(a, w, dis, cw, dtype, *, ls=None, bm=512):
    m_pad, k = a.shape
    n = w.shape[1]
    ncc = n // cw
    if ls is not None:
        a = a * ls[:, None]
    out = pl.pallas_call(
        functools.partial(_table_kernel, dtype=dtype),
        grid=(ncc, m_pad // bm),
        in_specs=[
            pl.BlockSpec((bm,), lambda c, i: (i,)),
            pl.BlockSpec((bm, k), lambda c, i: (i, 0)),
            pl.BlockSpec((k, cw), lambda c, i: (0, c)),
        ],
        out_specs=pl.BlockSpec((1, bm, cw), lambda c, i: (c, i, 0)),
        out_shape=jax.ShapeDtypeStruct((ncc, m_pad, cw), dtype),
    )(dis, a, w)
    return out


def _combine_kernel(dis_ref, p_ref, hs_ref, b_ref, o_ref, *,
                    bm, n_valid):
    dis = dis_ref[...][:, None]
    psum = (p_ref[0, 0, 0].astype(jnp.float32)
            + p_ref[1, 0, 0].astype(jnp.float32)
            + hs_ref[0].astype(jnp.float32))
    o = dis * psum + b_ref[0]
    o = jnp.maximum(o, 0.0)
    row = pl.program_id(1) * bm + lax.broadcasted_iota(jnp.int32, (bm, 1), 0)
    o_ref[...] = jnp.where(row < n_valid, o, 0.0)


def _combine(partials, hs, dis, b, n_valid, *, bm=512):
    ncc, m_pad, cw = hs.shape
    b2 = b.reshape(ncc, 1, cw)
    out = pl.pallas_call(
        functools.partial(_combine_kernel, bm=bm, n_valid=n_valid),
        grid=(ncc, m_pad // bm),
        in_specs=[
            pl.BlockSpec((bm,), lambda c, i: (i,)),
            pl.BlockSpec((2, 1, bm, cw), lambda c, i: (0, c, i, 0)),
            pl.BlockSpec((1, bm, cw), lambda c, i: (c, i, 0)),
            pl.BlockSpec((1, 1, cw), lambda c, i: (c, 0, 0)),
        ],
        out_specs=pl.BlockSpec((bm, cw), lambda c, i: (i, c)),
        out_shape=jax.ShapeDtypeStruct((m_pad, ncc * cw), jnp.float32),
    )(dis, partials, hs, b2)
    return out


def _sc_aggregate(table, src, dst, n_pad, ncc, cw):
    ep = src.shape[0]
    ept = ep // (_NC * _NS)
    nw = ept // _WIN
    rpt = n_pad // _NS
    nz = rpt // _ZB
    mesh = plsc.VectorSubcoreMesh(core_axis_name="c", subcore_axis_name="s")

    assert nw % 2 == 0 and nw >= 4
    @functools.partial(
        pl.kernel,
        out_type=jax.ShapeDtypeStruct((_NC, ncc, n_pad, cw), jnp.float32),
        mesh=mesh,
        scratch_types=[
            pltpu.VMEM((_WIN,), jnp.int32),
            pltpu.VMEM((_WIN,), jnp.int32),
            pltpu.VMEM((_WIN,), jnp.int32),
            pltpu.VMEM((_WIN,), jnp.int32),
            pltpu.VMEM((_WIN,), jnp.int32),
            pltpu.VMEM((_WIN,), jnp.int32),
            pltpu.VMEM((_WIN,), jnp.int32),
            pltpu.VMEM((_WIN,), jnp.int32),
            pltpu.VMEM((_WIN, cw), jnp.float32),
            pltpu.VMEM((_WIN, cw), jnp.float32),
            pltpu.VMEM_SHARED((n_pad, cw), jnp.float32),
            pltpu.SemaphoreType.DMA,
            pltpu.SemaphoreType.DMA,
            pltpu.SemaphoreType.DMA,
            pltpu.SemaphoreType.DMA,
            pltpu.SemaphoreType.DMA,
            pltpu.SemaphoreType.DMA,
        ],
    )
    def agg(table_h, src_h, dst_h, out_h, src_v0, dst_v0, srco_v0, sdst_v0,
            src_v1, dst_v1, srco_v1, sdst_v1, rows_v0, rows_v1,
            acc_s, isem0, isem1, gsem0, gsem1, ssem0, ssem1):
        c = lax.axis_index("c")
        s = lax.axis_index("s")
        tid = c * _NS + s
        zsplat = jnp.zeros((16,), jnp.float32)
        idx = ((src_v0, dst_v0, srco_v0, sdst_v0, isem0),
               (src_v1, dst_v1, srco_v1, sdst_v1, isem1))
        rows = (rows_v0, rows_v1)
        gsems = (gsem0, gsem1)
        ssems = (ssem0, ssem1)

        def zrow(r, _):
            for v in range(cw // 16):
                rows_v1[r, pl.ds(v * 16, 16)] = zsplat
            return 0

        def issue_idx(w, p):
            sv, dv = idx[p][0], idx[p][1]
            off = tid * ept + w * _WIN
            pltpu.async_copy(src_h.at[pl.ds(off, _WIN)], sv, idx[p][4])
            pltpu.async_copy(dst_h.at[pl.ds(off, _WIN)], dv, idx[p][4])

        def wait_idx(p):
            sv, dv = idx[p][0], idx[p][1]
            pltpu.make_async_copy(src_h.at[pl.ds(0, _WIN)], sv,
                                  idx[p][4]).wait()
            pltpu.make_async_copy(dst_h.at[pl.ds(0, _WIN)], dv,
                                  idx[p][4]).wait()

        def comp_srco(p, coff):
            sv, ov = idx[p][0], idx[p][2]
            for v in range(_WIN // 16):
                ov[pl.ds(v * 16, 16)] = sv[pl.ds(v * 16, 16)] + coff

        def copy_sdst(p):
            dv, sd = idx[p][1], idx[p][3]
            for v in range(_WIN // 16):
                sd[pl.ds(v * 16, 16)] = dv[pl.ds(v * 16, 16)]

        def issue_gather(p):
            pltpu.async_copy(table_h.at[idx[p][2]], rows[p], gsems[p])

        def wait_gather(p):
            pltpu.make_async_copy(table_h.at[idx[p][2]], rows[p],
                                  gsems[p]).wait()

        def issue_scat(p):
            pltpu.async_copy(rows[p], acc_s.at[idx[p][3]], ssems[p],
                             add=True)

        def wait_scat(p):
            pltpu.make_async_copy(rows[p], acc_s.at[idx[p][3]],
                                  ssems[p]).wait()

        def chunk_body(ch, _):
            lax.fori_loop(0, _ZB, zrow, 0)
            for j in range(nz):
                zoff = pl.multiple_of(s * rpt + j * _ZB, _ZB)
                pltpu.sync_copy(rows_v1.at[pl.ds(0, _ZB)],
                                acc_s.at[pl.ds(zoff, _ZB)])
            plsc.subcore_barrier()
            coff = ch * n_pad

            issue_idx(0, 0)
            wait_idx(0)
            comp_srco(0, coff)
            issue_gather(0)
            issue_idx(1, 1)

            def win_body(t, _):
                w0 = 2 * t

                wait_gather(0)
                copy_sdst(0)
                issue_scat(0)
                wait_idx(1)
                comp_srco(1, coff)

                @pl.when(w0 + 2 < nw)
                def _():
                    issue_idx(w0 + 2, 0)

                @pl.when(t > 0)
                def _():
                    wait_scat(1)
                issue_gather(1)

                wait_gather(1)
                copy_sdst(1)
                issue_scat(1)

                @pl.when(w0 + 2 < nw)
                def _():
                    wait_idx(0)
                    comp_srco(0, coff)

                @pl.when(w0 + 3 < nw)
                def _():
                    issue_idx(w0 + 3, 1)

                wait_scat(0)

                @pl.when(w0 + 2 < nw)
                def _():
                    issue_gather(0)

                return 0

            lax.fori_loop(0, nw // 2, win_body, 0)
            wait_scat(1)
            plsc.subcore_barrier()
            fl_v = rows_v0.at[pl.ds(0, _ZB)]
            for j in range(nz):
                foff = pl.multiple_of(s * rpt + j * _ZB, _ZB)
                pltpu.sync_copy(acc_s.at[pl.ds(foff, _ZB)], fl_v)
                pltpu.sync_copy(
                    fl_v, out_h.at[c, ch, pl.ds(foff, _ZB)])
            plsc.subcore_barrier()
            return 0

        lax.fori_loop(0, ncc, chunk_body, 0)

    return agg(table, src, dst)


def _sc_degree(dst, n_pad):
    ep = dst.shape[0]
    ept = ep // (_NC * _NS)
    nw = ept // _WIN
    rpt = n_pad // _NS
    nz = rpt // _ZB
    mesh = plsc.VectorSubcoreMesh(core_axis_name="c", subcore_axis_name="s")

    @functools.partial(
        pl.kernel,
        out_type=jax.ShapeDtypeStruct((_NC * n_pad,), jnp.float32),
        mesh=mesh,
        scratch_types=[
            pltpu.VMEM((_WIN,), jnp.int32),
            pltpu.VMEM((_WIN,), jnp.float32),
            pltpu.VMEM((_ZB,), jnp.float32),
            pltpu.VMEM((_ZB,), jnp.float32),
            pltpu.VMEM_SHARED((n_pad,), jnp.float32),
            pltpu.SemaphoreType.DMA,
        ],
    )
    def deg(dst_h, out_h, dst_v, ones_v, zeros_v, fl_v, acc_s, sem):
        c = lax.axis_index("c")
        s = lax.axis_index("s")
        tid = c * _NS + s
        for v in range(_WIN // 16):
            ones_v[pl.ds(v * 16, 16)] = jnp.ones((16,), jnp.float32)
        for v in range(_ZB // 16):
            zeros_v[pl.ds(v * 16, 16)] = jnp.zeros((16,), jnp.float32)
        for j in range(nz):
            pltpu.sync_copy(zeros_v, acc_s.at[pl.ds(s * rpt + j * _ZB, _ZB)])
        plsc.subcore_barrier()

        def win_body(w, _):
            off = tid * ept + w * _WIN
            pltpu.sync_copy(dst_h.at[pl.ds(off, _WIN)], dst_v)
            pltpu.sync_copy(ones_v, acc_s.at[dst_v], add=True)
            return 0

        lax.fori_loop(0, nw, win_body, 0)
        plsc.subcore_barrier()
        for j in range(nz):
            pltpu.sync_copy(acc_s.at[pl.ds(s * rpt + j * _ZB, _ZB)], fl_v)
            pltpu.sync_copy(
                fl_v, out_h.at[pl.ds(c * n_pad + s * rpt + j * _ZB, _ZB)])

    return deg(dst).reshape(_NC, n_pad)


def _gcn_layer(h, src, dst, dis, W, b, n, n_pad, cw, *, ls=None):
    cout = W.shape[1]
    if cout % cw:
        cpad = ((cout + cw - 1) // cw) * cw
        W = jnp.pad(W, ((0, 0), (0, cpad - cout)))
        b = jnp.pad(b, (0, cpad - cout))
    hs = _mm_table(h, W, dis, cw, jnp.float32, ls=ls)
    ncc = hs.shape[0]
    table = hs.reshape(ncc * n_pad, cw)
    partials = _sc_aggregate(table, src, dst, n_pad, ncc, cw)
    return _combine(partials, hs, dis, b, n)


def _sc_remap(mapping, src, dst, junk_base):
    n_map = mapping.shape[0]
    ep = src.shape[0]
    ept = ep // (_NC * _NS)
    nw = ept // _WIN
    mesh = plsc.VectorSubcoreMesh(core_axis_name="c", subcore_axis_name="s")

    @functools.partial(
        pl.kernel,
        out_type=(jax.ShapeDtypeStruct((ep,), jnp.int32),
                  jax.ShapeDtypeStruct((ep,), jnp.int32)),
        mesh=mesh,
        scratch_types=[
            pltpu.VMEM((_WIN,), jnp.int32),
            pltpu.VMEM((_WIN,), jnp.int32),
            pltpu.VMEM((_WIN,), jnp.int32),
            pltpu.VMEM((_WIN,), jnp.int32),
            pltpu.VMEM((_WIN,), jnp.int32),
            pltpu.VMEM((_WIN,), jnp.int32),
            pltpu.SemaphoreType.DMA,
        ],
    )
    def rm(map_h, src_h, dst_h, ns_h, nd_h, sv, dv, msv, mdv, nsv, ndv, sem):
        c = lax.axis_index("c")
        s = lax.axis_index("s")
        tid = c * _NS + s

        def win_body(w, _):
            lanes = lax.iota(jnp.int32, 16)
            off = tid * ept + w * _WIN
            pltpu.sync_copy(src_h.at[pl.ds(off, _WIN)], sv)
            pltpu.sync_copy(dst_h.at[pl.ds(off, _WIN)], dv)
            pltpu.async_copy(map_h.at[sv], msv, sem)
            pltpu.async_copy(map_h.at[dv], mdv, sem)
            pltpu.make_async_copy(map_h.at[sv], msv, sem).wait()
            pltpu.make_async_copy(map_h.at[dv], mdv, sem).wait()
            for v in range(_WIN // 16):
                ms = msv[pl.ds(v * 16, 16)]
                md = mdv[pl.ds(v * 16, 16)]
                ok = (ms >= 0) & (md >= 0)
                e = off + v * 16 + lanes
                junk = junk_base + (e & (_JUNK - 1))
                nsv[pl.ds(v * 16, 16)] = jnp.where(ok, ms, junk)
                ndv[pl.ds(v * 16, 16)] = jnp.where(ok, md, junk)
            pltpu.sync_copy(nsv, ns_h.at[pl.ds(off, _WIN)])
            pltpu.sync_copy(ndv, nd_h.at[pl.ds(off, _WIN)])
            return 0

        lax.fori_loop(0, nw, win_body, 0)

    return rm(mapping, src, dst)


def _pad_edges(src, dst, junk_base, ep):
    e = src.shape[0]
    pad = ep - e
    fill = junk_base + (jnp.arange(pad, dtype=jnp.int32) % _JUNK)
    return (jnp.concatenate([src, fill]), jnp.concatenate([dst, fill]))


def kernel(x, edge_index, W1, b1, W2, b2, W3, b3, p1, W4, b4, W5, b5, W6, b6,
           p2, Wlin, blin):
    n1 = x.shape[0]
    e = edge_index.shape[1]
    k1 = int(math.ceil(RATIO * n1))
    k2 = int(math.ceil(RATIO * k1))
    npad1 = ((n1 + _JUNK) + 1023) // 1024 * 1024
    npad2 = ((k1 + _JUNK) + 1023) // 1024 * 1024
    epm = _NC * _NS * _WIN * 2
    ep = ((e + epm - 1) // epm) * epm

    src0 = edge_index[0]
    dst0 = edge_index[1]
    src1, dst1 = _pad_edges(src0, dst0, n1, ep)

    degp1 = _sc_degree(dst1, npad1)
    deg1 = degp1[0] + degp1[1] + 1.0
    dis1 = lax.rsqrt(deg1)

    xp = _pad_rows(x, npad1)
    h = _gcn_layer(xp, src1, dst1, dis1, W1, b1, n1, npad1, 128)
    h = _gcn_layer(h, src1, dst1, dis1, W2, b2, n1, npad1, 128)
    h = _gcn_layer(h, src1, dst1, dis1, W3, b3, n1, npad1, 128)

    h_valid = h[:n1]
    score = jnp.tanh((h_valid @ p1) / jnp.linalg.norm(p1))
    vals, perm = lax.top_k(score, k1)
    x_new = h_valid[perm]
    mapping = jnp.full((npad1,), -1, jnp.int32).at[perm].set(
        jnp.arange(k1, dtype=jnp.int32))
    src2, dst2 = _sc_remap(mapping, src1, dst1, k1)

    degp2 = _sc_degree(dst2, npad2)
    deg2 = degp2[0] + degp2[1] + 1.0
    dis2 = lax.rsqrt(deg2)

    xg = _pad_rows(x_new, npad2)
    lsv = _pad_rows(vals[:, None], npad2)[:, 0]
    h = _gcn_layer(xg, src2, dst2, dis2, W4, b4, k1, npad2, 128, ls=lsv)
    h = _gcn_layer(h, src2, dst2, dis2, W5, b5, k1, npad2, 128)
    h = _gcn_layer(h, src2, dst2, dis2, W6, b6, k1, npad2, 128)

    h_valid = h[:k1]
    score2 = jnp.tanh((h_valid @ p2) / jnp.linalg.norm(p2))
    vals2, perm2 = lax.top_k(score2, k2)
    w_full = jnp.zeros((k1,), jnp.float32).at[perm2].set(vals2)
    g = (w_full @ h_valid)[None, :] / k2
    out = _matmul(g, Wlin) + blin
    return jax.nn.log_softmax(out, axis=1)
